# Initial kernel scaffold; baseline (speedup 1.0000x reference)
import jax
import jax.numpy as jnp
from jax.experimental import pallas as pl


def kernel(atoms, deg_slice, membership, deg_adj_1, deg_adj_2, deg_adj_3, deg_adj_4, deg_adj_5, deg_adj_6, W_stack, b_stack):
    raise NotImplementedError("write your pallas kernel here")


# speedup vs baseline: 2.1032x; 2.1032x over previous
import functools

import jax
import jax.numpy as jnp
from jax import lax
from jax.experimental import pallas as pl
from jax.experimental.pallas import tpu as pltpu
from jax.experimental.pallas import tpu_sc as plsc

MAX_DEG = 6
N_PER_DEG = 14000
N_ATOMS = (MAX_DEG + 1) * N_PER_DEG
FEAT = 128
BATCH = 64

NC = 2
NS = 16
NW = NC * NS
PAD_N = 14080
CHUNK = PAD_N // NW
SUB = 88
NSUB = CHUNK // SUB

BLK = 1000
NBLK = N_PER_DEG // BLK


def _sc_gather_body(atoms_hbm, idx_hbm, g1, g2, g3, g4, g5, g6,
                    idx_v, rows_v, sem):
    wid = lax.axis_index("s") * NC + lax.axis_index("c")
    base = wid * CHUNK
    outs = [g1, g2, g3, g4, g5, g6]
    for d in range(1, MAX_DEG + 1):
        off = d * (d - 1) // 2

        def body(s, _, d=d, off=off):
            rbase = (off + s) * PAD_N + base
            for c in range(NSUB):
                pltpu.sync_copy(idx_hbm.at[pl.ds(rbase + c * SUB, SUB)],
                                idx_v.at[c])
            cps = [pltpu.async_copy(atoms_hbm.at[idx_v.at[c]],
                                    rows_v.at[pl.ds(c * SUB, SUB)], sem)
                   for c in range(NSUB)]
            for cp in cps:
                cp.wait()
            pltpu.sync_copy(rows_v, outs[d - 1].at[s, pl.ds(base, CHUNK)])
            return _

        lax.fori_loop(0, d, body, None)


@functools.cache
def _make_sc_gather():
    return pl.kernel(
        _sc_gather_body,
        out_type=[jax.ShapeDtypeStruct((d, PAD_N, FEAT), jnp.float32)
                  for d in range(1, MAX_DEG + 1)],
        mesh=plsc.VectorSubcoreMesh(core_axis_name="c", subcore_axis_name="s",
                                    num_cores=NC, num_subcores=NS),
        scratch_types=[
            pltpu.VMEM((NSUB, SUB), jnp.int32),
            pltpu.VMEM((CHUNK, FEAT), jnp.float32),
            pltpu.SemaphoreType.DMA,
        ],
    )


def _sc_gather(atoms, idx_rows):
    return _make_sc_gather()(atoms, idx_rows)


def _tc_body(atoms_ref, g1, g2, g3, g4, g5, g6, wself, wrel, wgath,
             bact, bgath, mem_ref, act_out, gath_out):
    d = pl.program_id(0)
    j = pl.program_id(1)
    a = atoms_ref[...]

    gs = [g1, g2, g3, g4, g5, g6]
    ns = jnp.zeros_like(a)
    for dd in range(1, MAX_DEG + 1):
        ns = jnp.where(d == dd, jnp.sum(gs[dd - 1][...], axis=0), ns)

    act = (jnp.dot(ns, wrel[0], preferred_element_type=jnp.float32)
           + jnp.dot(a, wself[0], preferred_element_type=jnp.float32)
           + bact[0])
    act_out[...] = act

    g = jnp.dot(a, wgath[0], preferred_element_type=jnp.float32) + bgath[0]
    m = mem_ref[0, 0]
    onehot = (lax.broadcasted_iota(jnp.int32, (BATCH, BLK), 0)
              == m[None, :]).astype(jnp.float32)
    part = jnp.dot(onehot, g, preferred_element_type=jnp.float32)

    first = (d == 0) & (j == 0)

    @pl.when(first)
    def _():
        gath_out[...] = part

    @pl.when(jnp.logical_not(first))
    def _():
        gath_out[...] += part


def _tc_affine(atoms, gbufs, wself, wrel, wgath, bact, bgath, mem_r):
    g_specs = [
        pl.BlockSpec((dd, BLK, FEAT),
                     lambda d, j, dd=dd: (0, jnp.where(d == dd, j, 0), 0))
        for dd in range(1, MAX_DEG + 1)
    ]
    return pl.pallas_call(
        _tc_body,
        grid=(MAX_DEG + 1, NBLK),
        in_specs=[
            pl.BlockSpec((BLK, FEAT), lambda d, j: (d * NBLK + j, 0)),
            *g_specs,
            pl.BlockSpec((1, FEAT, FEAT), lambda d, j: (d, 0, 0)),
            pl.BlockSpec((1, FEAT, FEAT), lambda d, j: (d, 0, 0)),
            pl.BlockSpec((1, FEAT, FEAT), lambda d, j: (d, 0, 0)),
            pl.BlockSpec((1, 1, FEAT), lambda d, j: (d, 0, 0)),
            pl.BlockSpec((1, 1, FEAT), lambda d, j: (d, 0, 0)),
            pl.BlockSpec((1, 1, BLK), lambda d, j: (d * NBLK + j, 0, 0)),
        ],
        out_specs=[
            pl.BlockSpec((BLK, FEAT), lambda d, j: (d * NBLK + j, 0)),
            pl.BlockSpec((BATCH, FEAT), lambda d, j: (0, 0)),
        ],
        out_shape=[
            jax.ShapeDtypeStruct((N_ATOMS, FEAT), jnp.float32),
            jax.ShapeDtypeStruct((BATCH, FEAT), jnp.float32),
        ],
        compiler_params=pltpu.CompilerParams(
            dimension_semantics=("arbitrary", "arbitrary")),
    )(atoms, *gbufs, wself, wrel, wgath, bact, bgath, mem_r)


def kernel(atoms, deg_slice, membership, deg_adj_1, deg_adj_2, deg_adj_3,
           deg_adj_4, deg_adj_5, deg_adj_6, W_stack, b_stack):
    adjs = [deg_adj_1, deg_adj_2, deg_adj_3, deg_adj_4, deg_adj_5, deg_adj_6]
    idx_rows = jnp.concatenate([a.T for a in adjs], axis=0)
    idx_rows = jnp.pad(idx_rows, ((0, 0), (0, PAD_N - N_PER_DEG))).reshape(-1)

    gbufs = _sc_gather(atoms, idx_rows)

    i_self = jnp.array([12, 1, 3, 5, 7, 9, 11], dtype=jnp.int32)
    i_gath = jnp.array([19, 13, 14, 15, 16, 17, 18], dtype=jnp.int32)
    i_rel = jnp.array([0, 0, 2, 4, 6, 8, 10], dtype=jnp.int32)
    wself = W_stack[i_self]
    wgath = W_stack[i_gath]
    wrel = W_stack[i_rel].at[0].set(0.0)
    bact = (b_stack[i_self] + b_stack[i_rel].at[0].set(0.0)).reshape(
        MAX_DEG + 1, 1, FEAT)
    bgath = b_stack[i_gath].reshape(MAX_DEG + 1, 1, FEAT)
    mem_r = membership.reshape(N_ATOMS // BLK, 1, BLK)

    activated, atom_gather = _tc_affine(
        atoms, gbufs, wself, wrel, wgath, bact, bgath, mem_r)
    return activated, atom_gather
